# baseline (speedup 1.0000x reference)
import jax
import jax.numpy as jnp
from jax.experimental import pallas as pl
from jax.experimental.pallas import tpu as pltpu

_CHUNK = 1000
_NCHUNK = 10


def _linear_kernel(x_hbm, w_ref, b_ref, o_hbm,
                   xb0, xb1, ob0, ob1, ls0, ls1, ss0, ss1):
    xbufs = (xb0, xb1)
    obufs = (ob0, ob1)
    lsems = (ls0, ls1)
    ssems = (ss0, ss1)

    def load(i, slot):
        return pltpu.make_async_copy(
            x_hbm.at[pl.ds(i * _CHUNK, _CHUNK), :], xbufs[slot], lsems[slot])

    def store(i, slot):
        return pltpu.make_async_copy(
            obufs[slot], o_hbm.at[pl.ds(i * _CHUNK, _CHUNK), :], ssems[slot])

    w_scaled = 0.1 * w_ref[...]
    b_scaled = 0.1 * b_ref[...]

    load(0, 0).start()
    for i in range(_NCHUNK):
        slot = i % 2
        if i + 1 < _NCHUNK:
            load(i + 1, 1 - slot).start()
        load(i, slot).wait()
        if i >= 2:
            store(i - 2, slot).wait()
        x_blk = xbufs[slot][...]
        y = jax.lax.dot_general(
            x_blk, w_scaled,
            dimension_numbers=(((1,), (1,)), ((), ())),
            preferred_element_type=jnp.float32,
        )
        obufs[slot][...] = x_blk + (y + b_scaled)
        store(i, slot).start()
    store(_NCHUNK - 2, 0 if _NCHUNK % 2 == 0 else 1).wait()
    store(_NCHUNK - 1, 1 if _NCHUNK % 2 == 0 else 0).wait()


def kernel(x, W, b):
    m, a = x.shape
    b2d = b.reshape(1, a)
    return pl.pallas_call(
        _linear_kernel,
        in_specs=[
            pl.BlockSpec(memory_space=pl.ANY),
            pl.BlockSpec(memory_space=pltpu.MemorySpace.VMEM),
            pl.BlockSpec(memory_space=pltpu.MemorySpace.VMEM),
        ],
        out_specs=pl.BlockSpec(memory_space=pl.ANY),
        out_shape=jax.ShapeDtypeStruct((m, a), x.dtype),
        scratch_shapes=[
            pltpu.VMEM((_CHUNK, a), jnp.float32),
            pltpu.VMEM((_CHUNK, a), jnp.float32),
            pltpu.VMEM((_CHUNK, a), jnp.float32),
            pltpu.VMEM((_CHUNK, a), jnp.float32),
            pltpu.SemaphoreType.DMA,
            pltpu.SemaphoreType.DMA,
            pltpu.SemaphoreType.DMA,
            pltpu.SemaphoreType.DMA,
        ],
    )(x, W, b2d)


# speedup vs baseline: 1.2726x; 1.2726x over previous
import jax
import jax.numpy as jnp
from jax.experimental import pallas as pl
from jax.experimental.pallas import tpu as pltpu

_CHUNK = 2000
_NCHUNK = 5
_NSLOT = 3


def _linear_kernel(x_hbm, w_ref, b_ref, o_hbm, *rest):
    xbufs = rest[0:_NSLOT]
    obufs = rest[_NSLOT:2 * _NSLOT]
    lsems = rest[2 * _NSLOT:3 * _NSLOT]
    ssems = rest[3 * _NSLOT:4 * _NSLOT]

    def load(i):
        slot = i % _NSLOT
        return pltpu.make_async_copy(
            x_hbm.at[pl.ds(i * _CHUNK, _CHUNK), :], xbufs[slot], lsems[slot])

    def store(i):
        slot = i % _NSLOT
        return pltpu.make_async_copy(
            obufs[slot], o_hbm.at[pl.ds(i * _CHUNK, _CHUNK), :], ssems[slot])

    w_scaled = 0.1 * w_ref[...]
    b_scaled = 0.1 * b_ref[...]

    load(0).start()
    load(1).start()
    for i in range(_NCHUNK):
        if i + 2 < _NCHUNK:
            load(i + 2).start()
        load(i).wait()
        if i >= _NSLOT:
            store(i - _NSLOT).wait()
        x_blk = xbufs[i % _NSLOT][...]
        y = jax.lax.dot_general(
            x_blk, w_scaled,
            dimension_numbers=(((1,), (1,)), ((), ())),
            preferred_element_type=jnp.float32,
        )
        obufs[i % _NSLOT][...] = x_blk + (y + b_scaled)
        store(i).start()
    for i in range(max(0, _NCHUNK - _NSLOT), _NCHUNK):
        store(i).wait()


def kernel(x, W, b):
    m, a = x.shape
    b2d = b.reshape(1, a)
    return pl.pallas_call(
        _linear_kernel,
        in_specs=[
            pl.BlockSpec(memory_space=pl.ANY),
            pl.BlockSpec(memory_space=pltpu.MemorySpace.VMEM),
            pl.BlockSpec(memory_space=pltpu.MemorySpace.VMEM),
        ],
        out_specs=pl.BlockSpec(memory_space=pl.ANY),
        out_shape=jax.ShapeDtypeStruct((m, a), x.dtype),
        scratch_shapes=(
            [pltpu.VMEM((_CHUNK, a), jnp.float32)] * (2 * _NSLOT)
            + [pltpu.SemaphoreType.DMA] * (2 * _NSLOT)
        ),
    )(x, W, b2d)


# speedup vs baseline: 1.3384x; 1.0516x over previous
import jax
import jax.numpy as jnp
from jax.experimental import pallas as pl
from jax.experimental.pallas import tpu as pltpu

_CHUNK = 1000
_NCHUNK = 10


def _linear_kernel(x_hbm, w_ref, b_ref, o_hbm, *rest):
    bufs = rest[0:_NCHUNK]
    lsem = rest[_NCHUNK]
    ssem = rest[_NCHUNK + 1]

    def load(i):
        return pltpu.make_async_copy(
            x_hbm.at[pl.ds(i * _CHUNK, _CHUNK), :], bufs[i], lsem.at[i])

    def store(i):
        return pltpu.make_async_copy(
            bufs[i], o_hbm.at[pl.ds(i * _CHUNK, _CHUNK), :], ssem.at[i])

    for i in range(_NCHUNK):
        load(i).start()

    w_scaled = 0.1 * w_ref[...]
    b_scaled = 0.1 * b_ref[...]

    for i in range(_NCHUNK):
        load(i).wait()
        x_blk = bufs[i][...]
        y = jax.lax.dot_general(
            x_blk, w_scaled,
            dimension_numbers=(((1,), (1,)), ((), ())),
            preferred_element_type=jnp.float32,
        )
        bufs[i][...] = x_blk + (y + b_scaled)
        store(i).start()

    for i in range(_NCHUNK):
        store(i).wait()


def kernel(x, W, b):
    m, a = x.shape
    b2d = b.reshape(1, a)
    return pl.pallas_call(
        _linear_kernel,
        in_specs=[
            pl.BlockSpec(memory_space=pl.ANY),
            pl.BlockSpec(memory_space=pltpu.MemorySpace.VMEM),
            pl.BlockSpec(memory_space=pltpu.MemorySpace.VMEM),
        ],
        out_specs=pl.BlockSpec(memory_space=pl.ANY),
        out_shape=jax.ShapeDtypeStruct((m, a), x.dtype),
        scratch_shapes=(
            [pltpu.VMEM((_CHUNK, a), jnp.float32)] * _NCHUNK
            + [pltpu.SemaphoreType.DMA((_NCHUNK,)),
               pltpu.SemaphoreType.DMA((_NCHUNK,))]
        ),
    )(x, W, b2d)


# speedup vs baseline: 1.3395x; 1.0008x over previous
import jax
import jax.numpy as jnp
from jax.experimental import pallas as pl
from jax.experimental.pallas import tpu as pltpu

_CHUNK = 2000
_NCHUNK = 5


def _linear_kernel(x_hbm, w_ref, b_ref, o_hbm, *rest):
    bufs = rest[0:_NCHUNK]
    lsem = rest[_NCHUNK]
    ssem = rest[_NCHUNK + 1]

    def load(i):
        return pltpu.make_async_copy(
            x_hbm.at[pl.ds(i * _CHUNK, _CHUNK), :], bufs[i], lsem.at[i])

    def store(i):
        return pltpu.make_async_copy(
            bufs[i], o_hbm.at[pl.ds(i * _CHUNK, _CHUNK), :], ssem.at[i])

    for i in range(_NCHUNK):
        load(i).start()

    w_scaled = 0.1 * w_ref[...]
    b_scaled = 0.1 * b_ref[...]

    for i in range(_NCHUNK):
        load(i).wait()
        x_blk = bufs[i][...]
        y = jax.lax.dot_general(
            x_blk, w_scaled,
            dimension_numbers=(((1,), (1,)), ((), ())),
            preferred_element_type=jnp.float32,
        )
        bufs[i][...] = x_blk + (y + b_scaled)
        store(i).start()

    for i in range(_NCHUNK):
        store(i).wait()


def kernel(x, W, b):
    m, a = x.shape
    b2d = b.reshape(1, a)
    return pl.pallas_call(
        _linear_kernel,
        in_specs=[
            pl.BlockSpec(memory_space=pl.ANY),
            pl.BlockSpec(memory_space=pltpu.MemorySpace.VMEM),
            pl.BlockSpec(memory_space=pltpu.MemorySpace.VMEM),
        ],
        out_specs=pl.BlockSpec(memory_space=pl.ANY),
        out_shape=jax.ShapeDtypeStruct((m, a), x.dtype),
        scratch_shapes=(
            [pltpu.VMEM((_CHUNK, a), jnp.float32)] * _NCHUNK
            + [pltpu.SemaphoreType.DMA((_NCHUNK,)),
               pltpu.SemaphoreType.DMA((_NCHUNK,))]
        ),
    )(x, W, b2d)


# speedup vs baseline: 1.4158x; 1.0570x over previous
import jax
import jax.numpy as jnp
from jax.experimental import pallas as pl
from jax.experimental.pallas import tpu as pltpu

_SIZES = (2000, 2000, 2000, 2000, 1000, 504, 496)
_OFFS = tuple(sum(_SIZES[:i]) for i in range(len(_SIZES)))
_N = len(_SIZES)


def _linear_kernel(x_hbm, w_ref, b_ref, o_hbm, *rest):
    bufs = rest[0:_N]
    lsem = rest[_N]
    ssem = rest[_N + 1]

    def load(i):
        return pltpu.make_async_copy(
            x_hbm.at[pl.ds(_OFFS[i], _SIZES[i]), :], bufs[i], lsem.at[i])

    def store(i):
        return pltpu.make_async_copy(
            bufs[i], o_hbm.at[pl.ds(_OFFS[i], _SIZES[i]), :], ssem.at[i])

    for i in range(_N):
        load(i).start()

    w_scaled = 0.1 * w_ref[...]
    b_scaled = 0.1 * b_ref[...]

    for i in range(_N):
        load(i).wait()
        x_blk = bufs[i][...]
        y = jax.lax.dot_general(
            x_blk, w_scaled,
            dimension_numbers=(((1,), (1,)), ((), ())),
            preferred_element_type=jnp.float32,
        )
        bufs[i][...] = x_blk + (y + b_scaled)
        store(i).start()

    for i in range(_N):
        store(i).wait()


def kernel(x, W, b):
    m, a = x.shape
    b2d = b.reshape(1, a)
    return pl.pallas_call(
        _linear_kernel,
        in_specs=[
            pl.BlockSpec(memory_space=pl.ANY),
            pl.BlockSpec(memory_space=pltpu.MemorySpace.VMEM),
            pl.BlockSpec(memory_space=pltpu.MemorySpace.VMEM),
        ],
        out_specs=pl.BlockSpec(memory_space=pl.ANY),
        out_shape=jax.ShapeDtypeStruct((m, a), x.dtype),
        scratch_shapes=(
            [pltpu.VMEM((s, a), jnp.float32) for s in _SIZES]
            + [pltpu.SemaphoreType.DMA((_N,)),
               pltpu.SemaphoreType.DMA((_N,))]
        ),
    )(x, W, b2d)
